# baseline (speedup 1.0000x reference)
import jax
import jax.numpy as jnp
from jax import lax
from jax.experimental import pallas as pl
from jax.experimental.pallas import tpu as pltpu
from jax.experimental.pallas import tpu_sc as plsc

N = 64
B = 16
D = 512
NC = 2
NS = 16
NW = NC * NS
RPW = (B * D) // NW
G = 2
NBUF = 4
NG = RPW // G


def _row_ops(bufs, x_v, v, r, gg, iota):
    vs = jnp.full((16,), v, jnp.int32)
    gs = jnp.full((16,), gg, jnp.int32)

    def xload(off):
        return x_v[r, pl.ds(off, 16)]

    def gat(iv, jv, mask):
        return plsc.load_gather(bufs, [vs, gs, iv, jv], mask=mask)

    def sct(iv, jv, val, mask):
        plsc.store_scatter(bufs, [vs, gs, iv, jv], val, mask=mask)

    i1 = [iota + 16 * c for c in range(4)]
    w = [xload(16 * c) for c in range(4)]
    xlast = w[3]
    for c in range(4):
        sct(i1[c], i1[c], w[c], None)
    for m in range(1, 16):
        for c in range(3):
            i = i1[c]
            w[c] = jnp.maximum(w[c], xload(16 * c + m))
            sct(i, i + m, w[c], None)
        mask = iota <= 15 - m
        xs = xlast.at[jnp.minimum(iota + m, 15)].get(
            mode="promise_in_bounds"
        )
        w[3] = jnp.maximum(w[3], xs)
        sct(i1[3], i1[3] + m, w[3], mask)
    i2 = [2 * iota, 2 * iota + 32]
    w2 = []
    for c in range(2):
        i = i2[c]
        mask = None if c == 0 else (iota <= 7)
        a = gat(i, i + 15, mask)
        bq = gat(i + 1, i + 16, mask)
        cc = gat(i + 2, i + 17, mask)
        w2.append(jnp.maximum(jnp.maximum(a, bq), cc))
        sct(i, i + 17, w2[c], mask)
    for m in range(19, 32, 2):
        for c in range(2):
            i = i2[c]
            mask = None if c == 0 else (iota <= (31 - m) // 2)
            p2 = gat(i + 2, i + m, mask)
            w2[c] = jnp.maximum(w2[c], p2)
            sct(i, i + m, w2[c], mask)
    i4 = 4 * iota
    mask = iota <= 7
    a = gat(i4, i4 + 31, mask)
    bq = gat(i4 + 2, i4 + 33, mask)
    cc = gat(i4 + 4, i4 + 35, mask)
    w4 = jnp.maximum(jnp.maximum(a, bq), cc)
    sct(i4, i4 + 35, w4, mask)
    for m in range(39, 64, 4):
        mask = iota <= (63 - m) // 4
        p2 = gat(i4 + 4, i4 + m, mask)
        w4 = jnp.maximum(w4, p2)
        sct(i4, i4 + m, w4, mask)


def _sc_body(x_hbm, zeros_hbm, out_hbm, bufs, x_v, sems):
    cid = lax.axis_index("c")
    sid = lax.axis_index("s")
    wid = sid * NC + cid
    bw = wid // NC
    d0 = (wid % NC) * RPW
    pltpu.sync_copy(zeros_hbm, bufs)
    pltpu.sync_copy(x_hbm.at[bw].at[pl.ds(d0, RPW)], x_v)
    iota = lax.broadcasted_iota(jnp.int32, (16,), 0)

    def outer(it, carry):
        for v in range(NBUF):
            g = it * NBUF + v
            dd = d0 + g * G
            dst = out_hbm.at[bw].at[pl.ds(dd, G)]

            @pl.when(g >= NBUF)
            def _wait():
                pltpu.make_async_copy(bufs.at[v], dst, sems.at[v]).wait()

            def row(gg, c2):
                _row_ops(bufs, x_v, v, g * G + gg, gg, iota)
                return c2

            lax.fori_loop(0, G, row, 0)
            pltpu.make_async_copy(bufs.at[v], dst, sems.at[v]).start()
        return carry

    lax.fori_loop(0, NG // NBUF, outer, 0)
    for v in range(NBUF):
        dst = out_hbm.at[bw].at[pl.ds(d0, G)]
        pltpu.make_async_copy(bufs.at[v], dst, sems.at[v]).wait()


@jax.jit
def _run(x, zeros):
    mesh = plsc.VectorSubcoreMesh(
        core_axis_name="c", subcore_axis_name="s", num_cores=NC,
        num_subcores=NS,
    )
    fn = pl.kernel(
        _sc_body,
        out_type=jax.ShapeDtypeStruct((B, D, N, N), jnp.float32),
        mesh=mesh,
        scratch_types=[
            pltpu.VMEM((NBUF, G, N, N), jnp.float32),
            pltpu.VMEM((RPW, N), jnp.float32),
            pltpu.SemaphoreType.DMA((NBUF,)),
        ],
        compiler_params=pltpu.CompilerParams(needs_layout_passes=False),
    )
    return fn(x, zeros)


def kernel(x):
    zeros = jnp.zeros((NBUF, G, N, N), jnp.float32)
    return _run(x, zeros)


# speedup vs baseline: 1.0942x; 1.0942x over previous
import jax
import jax.numpy as jnp
from jax import lax
from jax.experimental import pallas as pl
from jax.experimental.pallas import tpu as pltpu
from jax.experimental.pallas import tpu_sc as plsc

N = 64
B = 16
D = 512
NC = 2
NS = 16
NW = NC * NS
RPW = (B * D) // NW
G = 4
NBUF = 2
NG = RPW // G


def _row_ops(bufs, x_v, v, r, gg, iota):
    vs = jnp.full((16,), v, jnp.int32)
    gs = jnp.full((16,), gg, jnp.int32)

    def xload(off):
        return x_v[r, pl.ds(off, 16)]

    def gat(iv, jv, mask):
        return plsc.load_gather(bufs, [vs, gs, iv, jv], mask=mask)

    def sct(iv, jv, val, mask):
        plsc.store_scatter(bufs, [vs, gs, iv, jv], val, mask=mask)

    i1 = [iota + 16 * c for c in range(4)]
    w = [xload(16 * c) for c in range(4)]
    xlast = w[3]
    for c in range(4):
        sct(i1[c], i1[c], w[c], None)
    for m in range(1, 16):
        for c in range(3):
            i = i1[c]
            w[c] = jnp.maximum(w[c], xload(16 * c + m))
            sct(i, i + m, w[c], None)
        mask = iota <= 15 - m
        xs = xlast.at[jnp.minimum(iota + m, 15)].get(
            mode="promise_in_bounds"
        )
        w[3] = jnp.maximum(w[3], xs)
        sct(i1[3], i1[3] + m, w[3], mask)
    i2 = [2 * iota, 2 * iota + 32]
    w2 = []
    for c in range(2):
        i = i2[c]
        mask = None if c == 0 else (iota <= 7)
        a = gat(i, i + 15, mask)
        bq = gat(i + 1, i + 16, mask)
        cc = gat(i + 2, i + 17, mask)
        w2.append(jnp.maximum(jnp.maximum(a, bq), cc))
        sct(i, i + 17, w2[c], mask)
    for m in range(19, 32, 2):
        for c in range(2):
            i = i2[c]
            mask = None if c == 0 else (iota <= (31 - m) // 2)
            p2 = gat(i + 2, i + m, mask)
            w2[c] = jnp.maximum(w2[c], p2)
            sct(i, i + m, w2[c], mask)
    i4 = 4 * iota
    mask = iota <= 7
    a = gat(i4, i4 + 31, mask)
    bq = gat(i4 + 2, i4 + 33, mask)
    cc = gat(i4 + 4, i4 + 35, mask)
    w4 = jnp.maximum(jnp.maximum(a, bq), cc)
    sct(i4, i4 + 35, w4, mask)
    for m in range(39, 64, 4):
        mask = iota <= (63 - m) // 4
        p2 = gat(i4 + 4, i4 + m, mask)
        w4 = jnp.maximum(w4, p2)
        sct(i4, i4 + m, w4, mask)


def _sc_body(x_hbm, zeros_hbm, out_hbm, bufs, x_v, sems):
    cid = lax.axis_index("c")
    sid = lax.axis_index("s")
    wid = sid * NC + cid
    bw = wid // NC
    d0 = (wid % NC) * RPW
    pltpu.sync_copy(zeros_hbm, bufs)
    pltpu.sync_copy(x_hbm.at[bw].at[pl.ds(d0, RPW)], x_v)
    iota = lax.broadcasted_iota(jnp.int32, (16,), 0)

    def outer(it, carry):
        for v in range(NBUF):
            g = it * NBUF + v
            dd = d0 + g * G
            dst = out_hbm.at[bw].at[pl.ds(dd, G)]

            @pl.when(g >= NBUF)
            def _wait():
                pltpu.make_async_copy(bufs.at[v], dst, sems.at[v]).wait()

            def row(gg, c2):
                _row_ops(bufs, x_v, v, g * G + gg, gg, iota)
                return c2

            lax.fori_loop(0, G, row, 0)
            pltpu.make_async_copy(bufs.at[v], dst, sems.at[v]).start()
        return carry

    lax.fori_loop(0, NG // NBUF, outer, 0)
    for v in range(NBUF):
        dst = out_hbm.at[bw].at[pl.ds(d0, G)]
        pltpu.make_async_copy(bufs.at[v], dst, sems.at[v]).wait()


@jax.jit
def _run(x, zeros):
    mesh = plsc.VectorSubcoreMesh(
        core_axis_name="c", subcore_axis_name="s", num_cores=NC,
        num_subcores=NS,
    )
    fn = pl.kernel(
        _sc_body,
        out_type=jax.ShapeDtypeStruct((B, D, N, N), jnp.float32),
        mesh=mesh,
        scratch_types=[
            pltpu.VMEM((NBUF, G, N, N), jnp.float32),
            pltpu.VMEM((RPW, N), jnp.float32),
            pltpu.SemaphoreType.DMA((NBUF,)),
        ],
        compiler_params=pltpu.CompilerParams(needs_layout_passes=False),
    )
    return fn(x, zeros)


def kernel(x):
    zeros = jnp.zeros((NBUF, G, N, N), jnp.float32)
    return _run(x, zeros)
